# baseline (speedup 1.0000x reference)
import jax
import jax.numpy as jnp
from jax import lax
from jax.experimental import pallas as pl
from jax.experimental.pallas import tpu as pltpu
from jax.experimental.pallas import tpu_sc as plsc

N = 10000
E = 160000
D = 256
K = 100

NC = 2
NS = 16
NP = 10240
LANES = 16

_CH = 80
_CE = 128
EP = NS * _CH * _CE
_WR = NP // NS


_DEG_CH = EP // (NC * NS) // 128


def _deg_body(dst2_hbm, out_hbm, idx_v, ones_v, zero_v, acc_sh, sem):
    c = lax.axis_index("c")
    s = lax.axis_index("s")

    def fill(i, _):
        ones_v[0, pl.ds(i * LANES, LANES)] = jnp.ones((LANES,), jnp.float32)
        return 0
    lax.fori_loop(0, 128 // LANES, fill, 0)

    def fillz(i, _):
        zero_v[pl.ds(i * LANES, LANES)] = jnp.zeros((LANES,), jnp.float32)
        return 0
    lax.fori_loop(0, _WR // LANES, fillz, 0)

    pltpu.sync_copy(zero_v, acc_sh.at[pl.ds(s * _WR, _WR)])

    w = c * NS + s
    pltpu.sync_copy(dst2_hbm.at[pl.ds(w * _DEG_CH, _DEG_CH)], idx_v)
    plsc.subcore_barrier()

    def chunk(k, _):
        pltpu.sync_copy(ones_v.at[0], acc_sh.at[idx_v.at[k]], add=True)
        return 0
    lax.fori_loop(0, _DEG_CH, chunk, 0)

    plsc.subcore_barrier()
    obase = pl.multiple_of(c * NP + s * _WR, 8)
    pltpu.sync_copy(acc_sh.at[pl.ds(s * _WR, _WR)],
                    out_hbm.at[pl.ds(obase, _WR)])


def _deg_call(dst2):
    mesh = plsc.VectorSubcoreMesh(core_axis_name="c", subcore_axis_name="s",
                                  num_cores=NC, num_subcores=NS)
    return pl.kernel(
        _deg_body,
        out_type=jax.ShapeDtypeStruct((NC * NP,), jnp.float32),
        mesh=mesh,
        scratch_types=[
            pltpu.VMEM((_DEG_CH, 128), jnp.int32),
            pltpu.VMEM((1, 128), jnp.float32),
            pltpu.VMEM((_WR,), jnp.float32),
            pltpu.VMEM_SHARED((NP,), jnp.float32),
            pltpu.SemaphoreType.DMA,
        ],
    )(dst2)


def _seg_body(src3_hbm, dst3_hbm, p4_hbm, z_hbm, out_hbm,
              isrc_v, idst_v, rows0_v, acc_sh, sem0):
    c = lax.axis_index("c")
    s = lax.axis_index("s")

    rbase = pl.multiple_of(s * _WR, 8)

    for j in range(2):
        q = 2 * c + j

        for z in range(5):
            pltpu.sync_copy(z_hbm, acc_sh.at[pl.ds(rbase + z * 128, 128)])
        plsc.subcore_barrier()

        crow = pl.multiple_of(s * _CH, 8)
        pltpu.sync_copy(src3_hbm.at[pl.ds(crow, _CH)], isrc_v)
        pltpu.sync_copy(dst3_hbm.at[pl.ds(crow, _CH)], idst_v)

        def chunk(k, _):
            pltpu.async_copy(p4_hbm.at[q].at[isrc_v.at[k]], rows0_v,
                             sem0).wait()
            pltpu.sync_copy(rows0_v, acc_sh.at[idst_v.at[k]], add=True)
            return 0
        lax.fori_loop(0, _CH, chunk, 0)

        plsc.subcore_barrier()

        @pl.when(s < NS - 1)
        def _w_full():
            for z in range(5):
                pltpu.sync_copy(
                    acc_sh.at[pl.ds(rbase + z * 128, 128)],
                    out_hbm.at[q].at[pl.ds(rbase + z * 128, 128)])

        @pl.when(s == NS - 1)
        def _w_last():
            for z in range(3):
                pltpu.sync_copy(
                    acc_sh.at[pl.ds(rbase + z * 128, 128)],
                    out_hbm.at[q].at[pl.ds(rbase + z * 128, 128)])
            pltpu.sync_copy(acc_sh.at[pl.ds(rbase + 384, 16)],
                            out_hbm.at[q].at[pl.ds(rbase + 384, 16)])

        plsc.subcore_barrier()


def _seg_call(src3, dst3, p4, z128):
    mesh = plsc.VectorSubcoreMesh(core_axis_name="c", subcore_axis_name="s",
                                  num_cores=NC, num_subcores=NS)
    return pl.kernel(
        _seg_body,
        out_type=jax.ShapeDtypeStruct((4, N, 128), jnp.float32),
        mesh=mesh,
        scratch_types=[
            pltpu.VMEM((_CH, _CE), jnp.int32),
            pltpu.VMEM((_CH, _CE), jnp.int32),
            pltpu.VMEM((_CE, 128), jnp.float32),
            pltpu.VMEM_SHARED((NP, 128), jnp.float32),
            pltpu.SemaphoreType.DMA,
        ],
    )(src3, dst3, p4, z128)


_BN = 1280


def _prep_body(x_ref, wg_ref, w1_ref, w2_ref, b2_ref, deg_ref,
               p4_ref, scal_ref):
    i = pl.program_id(0)
    xb = x_ref[...]
    h = jnp.dot(xb, wg_ref[...], preferred_element_type=jnp.float32)
    a = jnp.dot(xb, w1_ref[...], preferred_element_type=jnp.float32)
    b = jnp.dot(xb, w2_ref[...], preferred_element_type=jnp.float32)[:, 0]
    b = b + b2_ref[0]
    deg = (deg_ref[0, pl.ds(i * _BN, _BN)] + deg_ref[1, pl.ds(i * _BN, _BN)]
           + 1.0)
    dinv = lax.rsqrt(deg)
    na = jnp.sqrt(jnp.sum(a * a, axis=1))
    hh = h * dinv[:, None]
    ah = a / na[:, None]
    p4_ref[0] = hh[:, :128]
    p4_ref[1] = hh[:, 128:]
    p4_ref[2] = ah[:, :128]
    p4_ref[3] = ah[:, 128:]
    scal_ref[0, pl.ds(i * _BN, _BN)] = dinv
    scal_ref[1, pl.ds(i * _BN, _BN)] = b


def _prep_call(x, w_gcn, w1, w2, b2, deg2):
    grid = (NP // _BN,)
    return pl.pallas_call(
        _prep_body,
        grid=grid,
        in_specs=[
            pl.BlockSpec((_BN, D), lambda i: (i, 0)),
            pl.BlockSpec((D, D), lambda i: (0, 0)),
            pl.BlockSpec((D, D), lambda i: (0, 0)),
            pl.BlockSpec((D, 1), lambda i: (0, 0)),
            pl.BlockSpec(memory_space=pltpu.SMEM),
            pl.BlockSpec((NC, NP), lambda i: (0, 0)),
        ],
        out_specs=[
            pl.BlockSpec((4, _BN, 128), lambda i: (0, i, 0)),
            pl.BlockSpec((2, NP), lambda i: (0, 0)),
        ],
        out_shape=[
            jax.ShapeDtypeStruct((4, N, 128), jnp.float32),
            jax.ShapeDtypeStruct((2, NP), jnp.float32),
        ],
    )(x, w_gcn, w1, w2, b2, deg2)


_FN = NP // _BN


def _final_body(p4_ref, t4_ref, scal_ref, bg_ref, out_ref,
                fit_ref, xpool_ref, score_sm, idx_sm):
    i = pl.program_id(0)

    @pl.when(i < _FN)
    def _fitness():
        p = p4_ref[...]
        t = t4_ref[...]
        dinv = scal_ref[0, pl.ds(i * _BN, _BN)]
        b = scal_ref[1, pl.ds(i * _BN, _BN)]
        xp0 = dinv[:, None] * (t[0] + p[0]) + bg_ref[0, :128]
        xp1 = dinv[:, None] * (t[1] + p[1]) + bg_ref[0, 128:]
        xpool_ref[pl.ds(i * _BN, _BN), :128] = xp0
        xpool_ref[pl.ds(i * _BN, _BN), 128:] = xp1
        c = (jnp.sum(p[2] * t[2], axis=1) + jnp.sum(p[3] * t[3], axis=1)
             + 1.0)
        fit_ref[pl.ds(i * _BN, _BN)] = jax.nn.sigmoid(b * c)

    @pl.when(i == _FN)
    def _topk():
        rows = lax.broadcasted_iota(jnp.int32, (NP // 128, 128), 0)
        cols = lax.broadcasted_iota(jnp.int32, (NP // 128, 128), 1)
        idxm = rows * 128 + cols
        f0 = jnp.where(idxm < N, fit_ref[...].reshape(NP // 128, 128), -1.0)
        big = jnp.int32(2 ** 30)

        def pick(it, f):
            m = jnp.max(f)
            cand = jnp.where(f >= m, idxm, big)
            idx = jnp.min(cand)
            score_sm[it] = m
            idx_sm[it] = idx
            return jnp.where(idxm == idx, -1.0, f)
        lax.fori_loop(0, K, pick, f0)

        def emit(it, _):
            idx = idx_sm[it]
            sc = score_sm[it]
            out_ref[it, :] = xpool_ref[idx, :] * sc
            return 0
        lax.fori_loop(0, K, emit, 0)


def _final_call(p4, t4, scal, b_gcn):
    grid = (_FN + 1,)

    def blk(i):
        return (0, jnp.minimum(i, _FN - 1), 0)

    return pl.pallas_call(
        _final_body,
        grid=grid,
        in_specs=[
            pl.BlockSpec((4, _BN, 128), blk),
            pl.BlockSpec((4, _BN, 128), blk),
            pl.BlockSpec((2, NP), lambda i: (0, 0)),
            pl.BlockSpec((1, D), lambda i: (0, 0)),
        ],
        out_specs=pl.BlockSpec((K, D), lambda i: (0, 0)),
        out_shape=jax.ShapeDtypeStruct((K, D), jnp.float32),
        scratch_shapes=[
            pltpu.VMEM((NP,), jnp.float32),
            pltpu.VMEM((NP, D), jnp.float32),
            pltpu.SMEM((K,), jnp.float32),
            pltpu.SMEM((K,), jnp.int32),
        ],
        compiler_params=pltpu.CompilerParams(
            dimension_semantics=("arbitrary",)),
    )(p4, t4, scal, b_gcn.reshape(1, D))


@jax.jit
def kernel(x, edge_index, W_gcn, b_gcn, W1, W2, b2):
    src = edge_index[0]
    dst = edge_index[1]
    pad = EP - E
    srcp = jnp.concatenate([src, jnp.zeros((pad,), src.dtype)])
    dstp = jnp.concatenate([dst, jnp.full((pad,), N, dst.dtype)])
    dst2 = dstp.reshape(EP // 128, 128)
    src3 = srcp.reshape(NS * _CH, _CE)
    dst3 = dstp.reshape(NS * _CH, _CE)
    deg2 = _deg_call(dst2).reshape(NC, NP)
    p4, scal = _prep_call(x, W_gcn, W1, W2, b2, deg2)
    z128 = jnp.zeros((128, 128), jnp.float32)
    t4 = _seg_call(src3, dst3, p4, z128)
    return _final_call(p4, t4, scal, b_gcn)


# speedup vs baseline: 1.0091x; 1.0091x over previous
import jax
import jax.numpy as jnp
from jax import lax
from jax.experimental import pallas as pl
from jax.experimental.pallas import tpu as pltpu
from jax.experimental.pallas import tpu_sc as plsc

N = 10000
E = 160000
D = 256
K = 100

NC = 2
NS = 16
NP = 10240
LANES = 16

_CH = 80
_CE = 128
EP = NS * _CH * _CE
_WR = NP // NS


_DEG_CH = EP // (NC * NS) // 128


def _deg_body(dst2_hbm, out_hbm, idx_v, ones_v, zero_v, acc_sh, sem):
    c = lax.axis_index("c")
    s = lax.axis_index("s")

    def fill(i, _):
        ones_v[0, pl.ds(i * LANES, LANES)] = jnp.ones((LANES,), jnp.float32)
        return 0
    lax.fori_loop(0, 128 // LANES, fill, 0)

    def fillz(i, _):
        zero_v[pl.ds(i * LANES, LANES)] = jnp.zeros((LANES,), jnp.float32)
        return 0
    lax.fori_loop(0, _WR // LANES, fillz, 0)

    pltpu.sync_copy(zero_v, acc_sh.at[pl.ds(s * _WR, _WR)])

    w = c * NS + s
    pltpu.sync_copy(dst2_hbm.at[pl.ds(w * _DEG_CH, _DEG_CH)], idx_v)
    plsc.subcore_barrier()

    def chunk(k, _):
        pltpu.sync_copy(ones_v.at[0], acc_sh.at[idx_v.at[k]], add=True)
        return 0
    lax.fori_loop(0, _DEG_CH, chunk, 0)

    plsc.subcore_barrier()
    obase = pl.multiple_of(c * NP + s * _WR, 8)
    pltpu.sync_copy(acc_sh.at[pl.ds(s * _WR, _WR)],
                    out_hbm.at[pl.ds(obase, _WR)])


def _deg_call(dst2):
    mesh = plsc.VectorSubcoreMesh(core_axis_name="c", subcore_axis_name="s",
                                  num_cores=NC, num_subcores=NS)
    return pl.kernel(
        _deg_body,
        out_type=jax.ShapeDtypeStruct((NC * NP,), jnp.float32),
        mesh=mesh,
        scratch_types=[
            pltpu.VMEM((_DEG_CH, 128), jnp.int32),
            pltpu.VMEM((1, 128), jnp.float32),
            pltpu.VMEM((_WR,), jnp.float32),
            pltpu.VMEM_SHARED((NP,), jnp.float32),
            pltpu.SemaphoreType.DMA,
        ],
    )(dst2)


def _seg_body(src3_hbm, dst3_hbm, p4_hbm, z_hbm, out_hbm,
              isrc_v, idst_v, rows0_v, acc_sh, sem0):
    c = lax.axis_index("c")
    s = lax.axis_index("s")

    rbase = pl.multiple_of(s * _WR, 8)

    for j in range(2):
        q = 2 * c + j

        for z in range(5):
            pltpu.sync_copy(z_hbm.at[pl.ds(rbase + z * 128, 128)],
                            acc_sh.at[pl.ds(rbase + z * 128, 128)])
        plsc.subcore_barrier()

        crow = pl.multiple_of(s * _CH, 8)
        pltpu.sync_copy(src3_hbm.at[pl.ds(crow, _CH)], isrc_v)
        pltpu.sync_copy(dst3_hbm.at[pl.ds(crow, _CH)], idst_v)

        def chunk(k, _):
            pltpu.async_copy(p4_hbm.at[q].at[isrc_v.at[k]], rows0_v,
                             sem0).wait()
            pltpu.sync_copy(rows0_v, acc_sh.at[idst_v.at[k]], add=True)
            return 0
        lax.fori_loop(0, _CH, chunk, 0)

        plsc.subcore_barrier()

        @pl.when(s < NS - 1)
        def _w_full():
            for z in range(5):
                pltpu.sync_copy(
                    acc_sh.at[pl.ds(rbase + z * 128, 128)],
                    out_hbm.at[q].at[pl.ds(rbase + z * 128, 128)])

        @pl.when(s == NS - 1)
        def _w_last():
            for z in range(3):
                pltpu.sync_copy(
                    acc_sh.at[pl.ds(rbase + z * 128, 128)],
                    out_hbm.at[q].at[pl.ds(rbase + z * 128, 128)])
            pltpu.sync_copy(acc_sh.at[pl.ds(rbase + 384, 16)],
                            out_hbm.at[q].at[pl.ds(rbase + 384, 16)])

        plsc.subcore_barrier()


def _seg_call(src3, dst3, p4, z128):
    mesh = plsc.VectorSubcoreMesh(core_axis_name="c", subcore_axis_name="s",
                                  num_cores=NC, num_subcores=NS)
    return pl.kernel(
        _seg_body,
        out_type=jax.ShapeDtypeStruct((4, N, 128), jnp.float32),
        mesh=mesh,
        scratch_types=[
            pltpu.VMEM((_CH, _CE), jnp.int32),
            pltpu.VMEM((_CH, _CE), jnp.int32),
            pltpu.VMEM((_CE, 128), jnp.float32),
            pltpu.VMEM_SHARED((NP, 128), jnp.float32),
            pltpu.SemaphoreType.DMA,
        ],
    )(src3, dst3, p4, z128)


_BN = 1280


def _prep_body(x_ref, wg_ref, w1_ref, w2_ref, b2_ref, deg_ref,
               p4_ref, scal_ref):
    i = pl.program_id(0)
    xb = x_ref[...]
    h = jnp.dot(xb, wg_ref[...], preferred_element_type=jnp.float32)
    a = jnp.dot(xb, w1_ref[...], preferred_element_type=jnp.float32)
    b = jnp.dot(xb, w2_ref[...], preferred_element_type=jnp.float32)[:, 0]
    b = b + b2_ref[0]
    deg = (deg_ref[0, pl.ds(i * _BN, _BN)] + deg_ref[1, pl.ds(i * _BN, _BN)]
           + 1.0)
    dinv = lax.rsqrt(deg)
    na = jnp.sqrt(jnp.sum(a * a, axis=1))
    hh = h * dinv[:, None]
    ah = a / na[:, None]
    p4_ref[0] = hh[:, :128]
    p4_ref[1] = hh[:, 128:]
    p4_ref[2] = ah[:, :128]
    p4_ref[3] = ah[:, 128:]
    scal_ref[0, pl.ds(i * _BN, _BN)] = dinv
    scal_ref[1, pl.ds(i * _BN, _BN)] = b


def _prep_call(x, w_gcn, w1, w2, b2, deg2):
    grid = (NP // _BN,)
    return pl.pallas_call(
        _prep_body,
        grid=grid,
        in_specs=[
            pl.BlockSpec((_BN, D), lambda i: (i, 0)),
            pl.BlockSpec((D, D), lambda i: (0, 0)),
            pl.BlockSpec((D, D), lambda i: (0, 0)),
            pl.BlockSpec((D, 1), lambda i: (0, 0)),
            pl.BlockSpec(memory_space=pltpu.SMEM),
            pl.BlockSpec((NC, NP), lambda i: (0, 0)),
        ],
        out_specs=[
            pl.BlockSpec((4, _BN, 128), lambda i: (0, i, 0)),
            pl.BlockSpec((2, NP), lambda i: (0, 0)),
        ],
        out_shape=[
            jax.ShapeDtypeStruct((4, N, 128), jnp.float32),
            jax.ShapeDtypeStruct((2, NP), jnp.float32),
        ],
    )(x, w_gcn, w1, w2, b2, deg2)


_FN = NP // _BN


def _final_body(p4_ref, t4_ref, scal_ref, bg_ref, out_ref,
                fit_ref, xpool_ref, score_sm, idx_sm):
    i = pl.program_id(0)

    @pl.when(i < _FN)
    def _fitness():
        p = p4_ref[...]
        t = t4_ref[...]
        dinv = scal_ref[0, pl.ds(i * _BN, _BN)]
        b = scal_ref[1, pl.ds(i * _BN, _BN)]
        xp0 = dinv[:, None] * (t[0] + p[0]) + bg_ref[0, :128]
        xp1 = dinv[:, None] * (t[1] + p[1]) + bg_ref[0, 128:]
        xpool_ref[pl.ds(i * _BN, _BN), :128] = xp0
        xpool_ref[pl.ds(i * _BN, _BN), 128:] = xp1
        c = (jnp.sum(p[2] * t[2], axis=1) + jnp.sum(p[3] * t[3], axis=1)
             + 1.0)
        fit_ref[pl.ds(i * _BN, _BN)] = jax.nn.sigmoid(b * c)

    @pl.when(i == _FN)
    def _topk():
        rows = lax.broadcasted_iota(jnp.int32, (NP // 128, 128), 0)
        cols = lax.broadcasted_iota(jnp.int32, (NP // 128, 128), 1)
        idxm = rows * 128 + cols
        f0 = jnp.where(idxm < N, fit_ref[...].reshape(NP // 128, 128), -1.0)
        big = jnp.int32(2 ** 30)

        def pick(it, f):
            m = jnp.max(f)
            cand = jnp.where(f >= m, idxm, big)
            idx = jnp.min(cand)
            score_sm[it] = m
            idx_sm[it] = idx
            return jnp.where(idxm == idx, -1.0, f)
        lax.fori_loop(0, K, pick, f0)

        def emit(it, _):
            idx = idx_sm[it]
            sc = score_sm[it]
            out_ref[it, :] = xpool_ref[idx, :] * sc
            return 0
        lax.fori_loop(0, K, emit, 0)


def _final_call(p4, t4, scal, b_gcn):
    grid = (_FN + 1,)

    def blk(i):
        return (0, jnp.minimum(i, _FN - 1), 0)

    return pl.pallas_call(
        _final_body,
        grid=grid,
        in_specs=[
            pl.BlockSpec((4, _BN, 128), blk),
            pl.BlockSpec((4, _BN, 128), blk),
            pl.BlockSpec((2, NP), lambda i: (0, 0)),
            pl.BlockSpec((1, D), lambda i: (0, 0)),
        ],
        out_specs=pl.BlockSpec((K, D), lambda i: (0, 0)),
        out_shape=jax.ShapeDtypeStruct((K, D), jnp.float32),
        scratch_shapes=[
            pltpu.VMEM((NP,), jnp.float32),
            pltpu.VMEM((NP, D), jnp.float32),
            pltpu.SMEM((K,), jnp.float32),
            pltpu.SMEM((K,), jnp.int32),
        ],
        compiler_params=pltpu.CompilerParams(
            dimension_semantics=("arbitrary",)),
    )(p4, t4, scal, b_gcn.reshape(1, D))


@jax.jit
def kernel(x, edge_index, W_gcn, b_gcn, W1, W2, b2):
    src = edge_index[0]
    dst = edge_index[1]
    pad = EP - E
    srcp = jnp.concatenate([src, jnp.zeros((pad,), src.dtype)])
    dstp = jnp.concatenate([dst, jnp.full((pad,), N, dst.dtype)])
    dst2 = dstp.reshape(EP // 128, 128)
    src3 = srcp.reshape(NS * _CH, _CE)
    dst3 = dstp.reshape(NS * _CH, _CE)
    deg2 = _deg_call(dst2).reshape(NC, NP)
    p4, scal = _prep_call(x, W_gcn, W1, W2, b2, deg2)
    z128 = jnp.zeros((NP, 128), jnp.float32)
    t4 = _seg_call(src3, dst3, p4, z128)
    return _final_call(p4, t4, scal, b_gcn)


# speedup vs baseline: 1.5072x; 1.4936x over previous
import jax
import jax.numpy as jnp
from jax import lax
from jax.experimental import pallas as pl
from jax.experimental.pallas import tpu as pltpu
from jax.experimental.pallas import tpu_sc as plsc

N = 10000
E = 160000
D = 256
K = 100

NC = 2
NS = 16
NP = 10240
LANES = 16

_CH = 80
_CE = 128
EP = NS * _CH * _CE
_WR = NP // NS


_DEG_CH = EP // (NC * NS) // 128


def _deg_body(dst2_hbm, out_hbm, idx_v, ones_v, zero_v, acc_sh, sem):
    c = lax.axis_index("c")
    s = lax.axis_index("s")

    def fill(i, _):
        ones_v[0, pl.ds(i * LANES, LANES)] = jnp.ones((LANES,), jnp.float32)
        return 0
    lax.fori_loop(0, 128 // LANES, fill, 0)

    def fillz(i, _):
        zero_v[pl.ds(i * LANES, LANES)] = jnp.zeros((LANES,), jnp.float32)
        return 0
    lax.fori_loop(0, _WR // LANES, fillz, 0)

    pltpu.sync_copy(zero_v, acc_sh.at[pl.ds(s * _WR, _WR)])

    w = c * NS + s
    pltpu.sync_copy(dst2_hbm.at[pl.ds(w * _DEG_CH, _DEG_CH)], idx_v)
    plsc.subcore_barrier()

    def chunk(k, _):
        pltpu.sync_copy(ones_v.at[0], acc_sh.at[idx_v.at[k]], add=True)
        return 0
    lax.fori_loop(0, _DEG_CH, chunk, 0)

    plsc.subcore_barrier()
    obase = pl.multiple_of(c * NP + s * _WR, 8)
    pltpu.sync_copy(acc_sh.at[pl.ds(s * _WR, _WR)],
                    out_hbm.at[pl.ds(obase, _WR)])


def _deg_call(dst2):
    mesh = plsc.VectorSubcoreMesh(core_axis_name="c", subcore_axis_name="s",
                                  num_cores=NC, num_subcores=NS)
    return pl.kernel(
        _deg_body,
        out_type=jax.ShapeDtypeStruct((NC * NP,), jnp.float32),
        mesh=mesh,
        scratch_types=[
            pltpu.VMEM((_DEG_CH, 128), jnp.int32),
            pltpu.VMEM((1, 128), jnp.float32),
            pltpu.VMEM((_WR,), jnp.float32),
            pltpu.VMEM_SHARED((NP,), jnp.float32),
            pltpu.SemaphoreType.DMA,
        ],
    )(dst2)


_SEG_PER_TILE = E // NS
_SEG_CHUNK = 128
_SEG_FULL = _SEG_PER_TILE // _SEG_CHUNK
_SEG_TAIL = _SEG_PER_TILE - _SEG_FULL * _SEG_CHUNK
_SEG_ROWS = 624
_SEG_REM = N - NS * _SEG_ROWS


def _seg_body(src_hbm, dst_hbm, p4_hbm, out_hbm,
              isrc_v, idst_v, rows_v, zrows_v, acc_sh, sem):
    c = lax.axis_index("c")
    s = lax.axis_index("s")

    def fillz(i, _):
        zrows_v[i // 8, pl.ds((i % 8) * LANES, LANES)] = (
            jnp.zeros((LANES,), jnp.float32))
        return 0
    lax.fori_loop(0, 128 * 8, fillz, 0)

    ebase = pl.multiple_of(s * _SEG_PER_TILE, 8)
    rbase = pl.multiple_of(s * _SEG_ROWS, 8)

    for j in range(2):
        q = 2 * c + j

        for z in range(4):
            pltpu.sync_copy(zrows_v,
                            acc_sh.at[pl.ds(rbase + z * 128, 128)])
        pltpu.sync_copy(zrows_v.at[pl.ds(0, 112)],
                        acc_sh.at[pl.ds(rbase + 512, 112)])

        @pl.when(s == NS - 1)
        def _zrem():
            pltpu.sync_copy(zrows_v.at[pl.ds(0, _SEG_REM)],
                            acc_sh.at[pl.ds(NS * _SEG_ROWS, _SEG_REM)])
        plsc.subcore_barrier()

        def chunk(k, _):
            off = pl.multiple_of(ebase + k * _SEG_CHUNK, 8)
            pltpu.sync_copy(src_hbm.at[pl.ds(off, _SEG_CHUNK)], isrc_v.at[0])
            pltpu.sync_copy(dst_hbm.at[pl.ds(off, _SEG_CHUNK)], idst_v.at[0])
            pltpu.async_copy(p4_hbm.at[q].at[isrc_v.at[0]], rows_v, sem).wait()
            pltpu.sync_copy(rows_v, acc_sh.at[idst_v.at[0]], add=True)
            return 0
        lax.fori_loop(0, _SEG_FULL, chunk, 0)

        off = ebase + _SEG_FULL * _SEG_CHUNK
        pltpu.sync_copy(src_hbm.at[pl.ds(off, _SEG_TAIL)],
                        isrc_v.at[0, pl.ds(0, _SEG_TAIL)])
        pltpu.sync_copy(dst_hbm.at[pl.ds(off, _SEG_TAIL)],
                        idst_v.at[0, pl.ds(0, _SEG_TAIL)])
        pltpu.async_copy(p4_hbm.at[q].at[isrc_v.at[0, pl.ds(0, _SEG_TAIL)]],
                         rows_v.at[pl.ds(0, _SEG_TAIL)], sem).wait()
        pltpu.sync_copy(rows_v.at[pl.ds(0, _SEG_TAIL)],
                        acc_sh.at[idst_v.at[0, pl.ds(0, _SEG_TAIL)]], add=True)

        plsc.subcore_barrier()
        for z in range(4):
            pltpu.sync_copy(acc_sh.at[pl.ds(rbase + z * 128, 128)],
                            out_hbm.at[q].at[pl.ds(rbase + z * 128, 128)])
        pltpu.sync_copy(acc_sh.at[pl.ds(rbase + 512, 112)],
                        out_hbm.at[q].at[pl.ds(rbase + 512, 112)])

        @pl.when(s == NS - 1)
        def _wrem():
            pltpu.sync_copy(acc_sh.at[pl.ds(NS * _SEG_ROWS, _SEG_REM)],
                            out_hbm.at[q].at[pl.ds(NS * _SEG_ROWS, _SEG_REM)])
        plsc.subcore_barrier()


def _seg_call(src, dst, p4):
    mesh = plsc.VectorSubcoreMesh(core_axis_name="c", subcore_axis_name="s",
                                  num_cores=NC, num_subcores=NS)
    return pl.kernel(
        _seg_body,
        out_type=jax.ShapeDtypeStruct((4, N, 128), jnp.float32),
        mesh=mesh,
        scratch_types=[
            pltpu.VMEM((1, _SEG_CHUNK), jnp.int32),
            pltpu.VMEM((1, _SEG_CHUNK), jnp.int32),
            pltpu.VMEM((_SEG_CHUNK, 128), jnp.float32),
            pltpu.VMEM((_SEG_CHUNK, 128), jnp.float32),
            pltpu.VMEM_SHARED((N, 128), jnp.float32),
            pltpu.SemaphoreType.DMA,
        ],
    )(src, dst, p4)


_BN = 1280


def _prep_body(x_ref, wg_ref, w1_ref, w2_ref, b2_ref, deg_ref,
               p4_ref, scal_ref):
    i = pl.program_id(0)
    xb = x_ref[...]
    h = jnp.dot(xb, wg_ref[...], preferred_element_type=jnp.float32)
    a = jnp.dot(xb, w1_ref[...], preferred_element_type=jnp.float32)
    b = jnp.dot(xb, w2_ref[...], preferred_element_type=jnp.float32)[:, 0]
    b = b + b2_ref[0]
    deg = (deg_ref[0, pl.ds(i * _BN, _BN)] + deg_ref[1, pl.ds(i * _BN, _BN)]
           + 1.0)
    dinv = lax.rsqrt(deg)
    na = jnp.sqrt(jnp.sum(a * a, axis=1))
    hh = h * dinv[:, None]
    ah = a / na[:, None]
    p4_ref[0] = hh[:, :128]
    p4_ref[1] = hh[:, 128:]
    p4_ref[2] = ah[:, :128]
    p4_ref[3] = ah[:, 128:]
    scal_ref[0, pl.ds(i * _BN, _BN)] = dinv
    scal_ref[1, pl.ds(i * _BN, _BN)] = b


def _prep_call(x, w_gcn, w1, w2, b2, deg2):
    grid = (NP // _BN,)
    return pl.pallas_call(
        _prep_body,
        grid=grid,
        in_specs=[
            pl.BlockSpec((_BN, D), lambda i: (i, 0)),
            pl.BlockSpec((D, D), lambda i: (0, 0)),
            pl.BlockSpec((D, D), lambda i: (0, 0)),
            pl.BlockSpec((D, 1), lambda i: (0, 0)),
            pl.BlockSpec(memory_space=pltpu.SMEM),
            pl.BlockSpec((NC, NP), lambda i: (0, 0)),
        ],
        out_specs=[
            pl.BlockSpec((4, _BN, 128), lambda i: (0, i, 0)),
            pl.BlockSpec((2, NP), lambda i: (0, 0)),
        ],
        out_shape=[
            jax.ShapeDtypeStruct((4, N, 128), jnp.float32),
            jax.ShapeDtypeStruct((2, NP), jnp.float32),
        ],
    )(x, w_gcn, w1, w2, b2, deg2)


_FN = NP // _BN


def _final_body(p4_ref, t4_ref, scal_ref, bg_ref, out_ref,
                fit_ref, xpool_ref, score_sm, idx_sm):
    i = pl.program_id(0)

    @pl.when(i < _FN)
    def _fitness():
        p = p4_ref[...]
        t = t4_ref[...]
        dinv = scal_ref[0, pl.ds(i * _BN, _BN)]
        b = scal_ref[1, pl.ds(i * _BN, _BN)]
        xp0 = dinv[:, None] * (t[0] + p[0]) + bg_ref[0, :128]
        xp1 = dinv[:, None] * (t[1] + p[1]) + bg_ref[0, 128:]
        xpool_ref[pl.ds(i * _BN, _BN), :128] = xp0
        xpool_ref[pl.ds(i * _BN, _BN), 128:] = xp1
        c = (jnp.sum(p[2] * t[2], axis=1) + jnp.sum(p[3] * t[3], axis=1)
             + 1.0)
        fit_ref[pl.ds(i * _BN, _BN)] = jax.nn.sigmoid(b * c)

    @pl.when(i == _FN)
    def _topk():
        rows = lax.broadcasted_iota(jnp.int32, (NP // 128, 128), 0)
        cols = lax.broadcasted_iota(jnp.int32, (NP // 128, 128), 1)
        idxm = rows * 128 + cols
        f0 = jnp.where(idxm < N, fit_ref[...].reshape(NP // 128, 128), -1.0)
        big = jnp.int32(2 ** 30)

        def pick(it, f):
            m = jnp.max(f)
            cand = jnp.where(f >= m, idxm, big)
            idx = jnp.min(cand)
            score_sm[it] = m
            idx_sm[it] = idx
            return jnp.where(idxm == idx, -1.0, f)
        lax.fori_loop(0, K, pick, f0)

        def emit(it, _):
            idx = idx_sm[it]
            sc = score_sm[it]
            out_ref[it, :] = xpool_ref[idx, :] * sc
            return 0
        lax.fori_loop(0, K, emit, 0)


def _final_call(p4, t4, scal, b_gcn):
    grid = (_FN + 1,)

    def blk(i):
        return (0, jnp.minimum(i, _FN - 1), 0)

    return pl.pallas_call(
        _final_body,
        grid=grid,
        in_specs=[
            pl.BlockSpec((4, _BN, 128), blk),
            pl.BlockSpec((4, _BN, 128), blk),
            pl.BlockSpec((2, NP), lambda i: (0, 0)),
            pl.BlockSpec((1, D), lambda i: (0, 0)),
        ],
        out_specs=pl.BlockSpec((K, D), lambda i: (0, 0)),
        out_shape=jax.ShapeDtypeStruct((K, D), jnp.float32),
        scratch_shapes=[
            pltpu.VMEM((NP,), jnp.float32),
            pltpu.VMEM((NP, D), jnp.float32),
            pltpu.SMEM((K,), jnp.float32),
            pltpu.SMEM((K,), jnp.int32),
        ],
        compiler_params=pltpu.CompilerParams(
            dimension_semantics=("arbitrary",)),
    )(p4, t4, scal, b_gcn.reshape(1, D))


@jax.jit
def kernel(x, edge_index, W_gcn, b_gcn, W1, W2, b2):
    src = edge_index[0]
    dst = edge_index[1]
    pad = EP - E
    srcp = jnp.concatenate([src, jnp.zeros((pad,), src.dtype)])
    dstp = jnp.concatenate([dst, jnp.full((pad,), N, dst.dtype)])
    dst2 = dstp.reshape(EP // 128, 128)
    deg2 = _deg_call(dst2).reshape(NC, NP)
    p4, scal = _prep_call(x, W_gcn, W1, W2, b2, deg2)
    t4 = _seg_call(src, dst, p4)
    return _final_call(p4, t4, scal, b_gcn)


# speedup vs baseline: 2.2147x; 1.4695x over previous
import jax
import jax.numpy as jnp
from jax import lax
from jax.experimental import pallas as pl
from jax.experimental.pallas import tpu as pltpu
from jax.experimental.pallas import tpu_sc as plsc

N = 10000
E = 160000
D = 256
K = 100

NC = 2
NS = 16
NP = 10240
LANES = 16

_CH = 80
_CE = 128
EP = NS * _CH * _CE
_WR = NP // NS


_DEG_CH = EP // (NC * NS) // 128


def _deg_body(dst2_hbm, out_hbm, idx_v, ones_v, zero_v, acc_sh, sem):
    c = lax.axis_index("c")
    s = lax.axis_index("s")

    def fill(i, _):
        ones_v[0, pl.ds(i * LANES, LANES)] = jnp.ones((LANES,), jnp.float32)
        return 0
    lax.fori_loop(0, 128 // LANES, fill, 0)

    def fillz(i, _):
        zero_v[pl.ds(i * LANES, LANES)] = jnp.zeros((LANES,), jnp.float32)
        return 0
    lax.fori_loop(0, _WR // LANES, fillz, 0)

    pltpu.sync_copy(zero_v, acc_sh.at[pl.ds(s * _WR, _WR)])

    w = c * NS + s
    pltpu.sync_copy(dst2_hbm.at[pl.ds(w * _DEG_CH, _DEG_CH)], idx_v)
    plsc.subcore_barrier()

    def chunk(k, _):
        pltpu.sync_copy(ones_v.at[0], acc_sh.at[idx_v.at[k]], add=True)
        return 0
    lax.fori_loop(0, _DEG_CH, chunk, 0)

    plsc.subcore_barrier()
    obase = pl.multiple_of(c * NP + s * _WR, 8)
    pltpu.sync_copy(acc_sh.at[pl.ds(s * _WR, _WR)],
                    out_hbm.at[pl.ds(obase, _WR)])


def _deg_call(dst2):
    mesh = plsc.VectorSubcoreMesh(core_axis_name="c", subcore_axis_name="s",
                                  num_cores=NC, num_subcores=NS)
    return pl.kernel(
        _deg_body,
        out_type=jax.ShapeDtypeStruct((NC * NP,), jnp.float32),
        mesh=mesh,
        scratch_types=[
            pltpu.VMEM((_DEG_CH, 128), jnp.int32),
            pltpu.VMEM((1, 128), jnp.float32),
            pltpu.VMEM((_WR,), jnp.float32),
            pltpu.VMEM_SHARED((NP,), jnp.float32),
            pltpu.SemaphoreType.DMA,
        ],
    )(dst2)


_SEG_PER_TILE = E // NS
_SEG_CHUNK = 128
_SEG_FULL = _SEG_PER_TILE // _SEG_CHUNK
_SEG_TAIL = _SEG_PER_TILE - _SEG_FULL * _SEG_CHUNK
_SEG_ROWS = 624
_SEG_REM = N - NS * _SEG_ROWS


def _seg_body(src_hbm, dst_hbm, p4_hbm, out_hbm,
              isrc0_v, idst0_v, isrc1_v, idst1_v,
              rows0_v, rows1_v, zrows_v, acc_sh, sem0, sem1):
    c = lax.axis_index("c")
    s = lax.axis_index("s")
    isrcs = (isrc0_v, isrc1_v)
    idsts = (idst0_v, idst1_v)
    rows_bufs = (rows0_v, rows1_v)
    sems = (sem0, sem1)

    def fillz(i, _):
        zrows_v[i // 8, pl.ds((i % 8) * LANES, LANES)] = (
            jnp.zeros((LANES,), jnp.float32))
        return 0
    lax.fori_loop(0, 128 * 8, fillz, 0)

    ebase = pl.multiple_of(s * _SEG_PER_TILE, 8)
    rbase = pl.multiple_of(s * _SEG_ROWS, 8)

    for j in range(2):
        q = 2 * c + j

        for z in range(4):
            pltpu.sync_copy(zrows_v,
                            acc_sh.at[pl.ds(rbase + z * 128, 128)])
        pltpu.sync_copy(zrows_v.at[pl.ds(0, 112)],
                        acc_sh.at[pl.ds(rbase + 512, 112)])

        @pl.when(s == NS - 1)
        def _zrem():
            pltpu.sync_copy(zrows_v.at[pl.ds(0, _SEG_REM)],
                            acc_sh.at[pl.ds(NS * _SEG_ROWS, _SEG_REM)])
        plsc.subcore_barrier()

        for b in range(2):
            offp = pl.multiple_of(ebase + b * _SEG_CHUNK, 8)
            pltpu.sync_copy(src_hbm.at[pl.ds(offp, _SEG_CHUNK)],
                            isrcs[b].at[0])
            pltpu.sync_copy(dst_hbm.at[pl.ds(offp, _SEG_CHUNK)],
                            idsts[b].at[0])
        pltpu.async_copy(p4_hbm.at[q].at[isrc0_v.at[0]], rows0_v, sem0)

        def pair(k2, _):
            for b in range(2):
                k = k2 * 2 + b
                pltpu.make_async_copy(p4_hbm.at[q].at[isrcs[b].at[0]],
                                      rows_bufs[b], sems[b]).wait()

                @pl.when(k + 1 < _SEG_FULL)
                def _fire():
                    pltpu.async_copy(p4_hbm.at[q].at[isrcs[1 - b].at[0]],
                                     rows_bufs[1 - b], sems[1 - b])

                pltpu.sync_copy(rows_bufs[b], acc_sh.at[idsts[b].at[0]],
                                add=True)

                @pl.when(k + 2 < _SEG_FULL)
                def _preload():
                    off2 = pl.multiple_of(ebase + (k + 2) * _SEG_CHUNK, 8)
                    pltpu.sync_copy(src_hbm.at[pl.ds(off2, _SEG_CHUNK)],
                                    isrcs[b].at[0])
                    pltpu.sync_copy(dst_hbm.at[pl.ds(off2, _SEG_CHUNK)],
                                    idsts[b].at[0])
            return 0
        lax.fori_loop(0, _SEG_FULL // 2, pair, 0)

        off = ebase + _SEG_FULL * _SEG_CHUNK
        pltpu.sync_copy(src_hbm.at[pl.ds(off, _SEG_TAIL)],
                        isrc0_v.at[0, pl.ds(0, _SEG_TAIL)])
        pltpu.sync_copy(dst_hbm.at[pl.ds(off, _SEG_TAIL)],
                        idst0_v.at[0, pl.ds(0, _SEG_TAIL)])
        pltpu.async_copy(p4_hbm.at[q].at[isrc0_v.at[0, pl.ds(0, _SEG_TAIL)]],
                         rows0_v.at[pl.ds(0, _SEG_TAIL)], sem0).wait()
        pltpu.sync_copy(rows0_v.at[pl.ds(0, _SEG_TAIL)],
                        acc_sh.at[idst0_v.at[0, pl.ds(0, _SEG_TAIL)]],
                        add=True)

        plsc.subcore_barrier()
        for z in range(4):
            pltpu.sync_copy(acc_sh.at[pl.ds(rbase + z * 128, 128)],
                            out_hbm.at[q].at[pl.ds(rbase + z * 128, 128)])
        pltpu.sync_copy(acc_sh.at[pl.ds(rbase + 512, 112)],
                        out_hbm.at[q].at[pl.ds(rbase + 512, 112)])

        @pl.when(s == NS - 1)
        def _wrem():
            pltpu.sync_copy(acc_sh.at[pl.ds(NS * _SEG_ROWS, _SEG_REM)],
                            out_hbm.at[q].at[pl.ds(NS * _SEG_ROWS, _SEG_REM)])
        plsc.subcore_barrier()


def _seg_call(src, dst, p4):
    mesh = plsc.VectorSubcoreMesh(core_axis_name="c", subcore_axis_name="s",
                                  num_cores=NC, num_subcores=NS)
    return pl.kernel(
        _seg_body,
        out_type=jax.ShapeDtypeStruct((4, N, 128), jnp.float32),
        mesh=mesh,
        scratch_types=[
            pltpu.VMEM((1, _SEG_CHUNK), jnp.int32),
            pltpu.VMEM((1, _SEG_CHUNK), jnp.int32),
            pltpu.VMEM((1, _SEG_CHUNK), jnp.int32),
            pltpu.VMEM((1, _SEG_CHUNK), jnp.int32),
            pltpu.VMEM((_SEG_CHUNK, 128), jnp.float32),
            pltpu.VMEM((_SEG_CHUNK, 128), jnp.float32),
            pltpu.VMEM((_SEG_CHUNK, 128), jnp.float32),
            pltpu.VMEM_SHARED((N, 128), jnp.float32),
            pltpu.SemaphoreType.DMA,
            pltpu.SemaphoreType.DMA,
        ],
    )(src, dst, p4)


_BN = 1280


def _prep_body(x_ref, wg_ref, w1_ref, w2_ref, b2_ref, deg_ref,
               p4_ref, scal_ref):
    i = pl.program_id(0)
    xb = x_ref[...]
    h = jnp.dot(xb, wg_ref[...], preferred_element_type=jnp.float32)
    a = jnp.dot(xb, w1_ref[...], preferred_element_type=jnp.float32)
    b = jnp.dot(xb, w2_ref[...], preferred_element_type=jnp.float32)[:, 0]
    b = b + b2_ref[0]
    deg = (deg_ref[0, pl.ds(i * _BN, _BN)] + deg_ref[1, pl.ds(i * _BN, _BN)]
           + 1.0)
    dinv = lax.rsqrt(deg)
    na = jnp.sqrt(jnp.sum(a * a, axis=1))
    hh = h * dinv[:, None]
    ah = a / na[:, None]
    p4_ref[0] = hh[:, :128]
    p4_ref[1] = hh[:, 128:]
    p4_ref[2] = ah[:, :128]
    p4_ref[3] = ah[:, 128:]
    scal_ref[0, pl.ds(i * _BN, _BN)] = dinv
    scal_ref[1, pl.ds(i * _BN, _BN)] = b


def _prep_call(x, w_gcn, w1, w2, b2, deg2):
    grid = (NP // _BN,)
    return pl.pallas_call(
        _prep_body,
        grid=grid,
        in_specs=[
            pl.BlockSpec((_BN, D), lambda i: (i, 0)),
            pl.BlockSpec((D, D), lambda i: (0, 0)),
            pl.BlockSpec((D, D), lambda i: (0, 0)),
            pl.BlockSpec((D, 1), lambda i: (0, 0)),
            pl.BlockSpec(memory_space=pltpu.SMEM),
            pl.BlockSpec((NC, NP), lambda i: (0, 0)),
        ],
        out_specs=[
            pl.BlockSpec((4, _BN, 128), lambda i: (0, i, 0)),
            pl.BlockSpec((2, NP), lambda i: (0, 0)),
        ],
        out_shape=[
            jax.ShapeDtypeStruct((4, N, 128), jnp.float32),
            jax.ShapeDtypeStruct((2, NP), jnp.float32),
        ],
    )(x, w_gcn, w1, w2, b2, deg2)


_FN = NP // _BN


def _final_body(p4_ref, t4_ref, scal_ref, bg_ref, out_ref,
                fit_ref, xpool_ref, score_sm, idx_sm):
    i = pl.program_id(0)

    @pl.when(i < _FN)
    def _fitness():
        p = p4_ref[...]
        t = t4_ref[...]
        dinv = scal_ref[0, pl.ds(i * _BN, _BN)]
        b = scal_ref[1, pl.ds(i * _BN, _BN)]
        xp0 = dinv[:, None] * (t[0] + p[0]) + bg_ref[0, :128]
        xp1 = dinv[:, None] * (t[1] + p[1]) + bg_ref[0, 128:]
        xpool_ref[pl.ds(i * _BN, _BN), :128] = xp0
        xpool_ref[pl.ds(i * _BN, _BN), 128:] = xp1
        c = (jnp.sum(p[2] * t[2], axis=1) + jnp.sum(p[3] * t[3], axis=1)
             + 1.0)
        fit_ref[pl.ds(i * _BN, _BN)] = jax.nn.sigmoid(b * c)

    @pl.when(i == _FN)
    def _topk():
        rows = lax.broadcasted_iota(jnp.int32, (NP // 128, 128), 0)
        cols = lax.broadcasted_iota(jnp.int32, (NP // 128, 128), 1)
        idxm = rows * 128 + cols
        f0 = jnp.where(idxm < N, fit_ref[...].reshape(NP // 128, 128), -1.0)
        big = jnp.int32(2 ** 30)

        def pick(it, f):
            m = jnp.max(f)
            cand = jnp.where(f >= m, idxm, big)
            idx = jnp.min(cand)
            score_sm[it] = m
            idx_sm[it] = idx
            return jnp.where(idxm == idx, -1.0, f)
        lax.fori_loop(0, K, pick, f0)

        def emit(it, _):
            idx = idx_sm[it]
            sc = score_sm[it]
            out_ref[it, :] = xpool_ref[idx, :] * sc
            return 0
        lax.fori_loop(0, K, emit, 0)


def _final_call(p4, t4, scal, b_gcn):
    grid = (_FN + 1,)

    def blk(i):
        return (0, jnp.minimum(i, _FN - 1), 0)

    return pl.pallas_call(
        _final_body,
        grid=grid,
        in_specs=[
            pl.BlockSpec((4, _BN, 128), blk),
            pl.BlockSpec((4, _BN, 128), blk),
            pl.BlockSpec((2, NP), lambda i: (0, 0)),
            pl.BlockSpec((1, D), lambda i: (0, 0)),
        ],
        out_specs=pl.BlockSpec((K, D), lambda i: (0, 0)),
        out_shape=jax.ShapeDtypeStruct((K, D), jnp.float32),
        scratch_shapes=[
            pltpu.VMEM((NP,), jnp.float32),
            pltpu.VMEM((NP, D), jnp.float32),
            pltpu.SMEM((K,), jnp.float32),
            pltpu.SMEM((K,), jnp.int32),
        ],
        compiler_params=pltpu.CompilerParams(
            dimension_semantics=("arbitrary",)),
    )(p4, t4, scal, b_gcn.reshape(1, D))


@jax.jit
def kernel(x, edge_index, W_gcn, b_gcn, W1, W2, b2):
    src = edge_index[0]
    dst = edge_index[1]
    dstp = jnp.concatenate([dst, jnp.full((EP - E,), N, dst.dtype)])
    dst2 = dstp.reshape(EP // 128, 128)
    deg2 = _deg_call(dst2).reshape(NC, NP)
    p4, scal = _prep_call(x, W_gcn, W1, W2, b2, deg2)
    t4 = _seg_call(src, dst, p4)
    return _final_call(p4, t4, scal, b_gcn)
